# Initial kernel scaffold; baseline (speedup 1.0000x reference)
import jax
import jax.numpy as jnp
from jax.experimental import pallas as pl


def kernel(input, params):
    raise NotImplementedError("write your pallas kernel here")


# speedup vs baseline: 2.4512x; 2.4512x over previous
import functools
from functools import partial

import jax
import jax.numpy as jnp
import numpy as np
from jax.experimental import pallas as pl

K_NN = 16
BIG = 3.0e38

_DOT = partial(jax.lax.dot_general, precision=jax.lax.Precision.HIGHEST,
               preferred_element_type=jnp.float32)


def _bmm(a, b):
    return jax.lax.dot_general(a.astype(jnp.bfloat16), b.astype(jnp.bfloat16),
                               (((1,), (0,)), ((), ())),
                               preferred_element_type=jnp.float32)


def _knn_feats_body(c_ref, out_ref, *, T, N):
    t = pl.program_id(1)
    c = c_ref[0]
    q = c_ref[0, pl.ds(t * T, T), :]

    cross = jax.lax.dot_general(q.astype(jnp.bfloat16), c.astype(jnp.bfloat16),
                                (((1,), (1,)), ((), ())),
                                preferred_element_type=jnp.float32)
    q0, q1, q2 = q[:, 0:1], q[:, 1:2], q[:, 2:3]
    n2q = (q0 * q0 + q1 * q1) + q2 * q2
    n2c = _DOT(jnp.ones((1, 3), jnp.float32), c * c,
               (((1,), (1,)), ((), ())))
    d2 = (n2q + n2c) - 2.0 * cross

    col = jax.lax.broadcasted_iota(jnp.int32, (T, N), 1)

    def step(j, d2m):
        m = jnp.min(d2m, axis=1, keepdims=True)
        idxc = jnp.where(d2m == m, col, N)
        im = jnp.min(idxc, axis=1, keepdims=True)
        oh = (col == im).astype(jnp.float32)
        nb = _DOT(oh, c, (((1,), (0,)), ((), ())))
        dist = jnp.maximum(m, 0.0)
        feats = jnp.concatenate([q, nb, q - nb, dist], axis=1)

        @pl.when(j > 0)
        def _():
            out_ref[0, jnp.maximum(j - 1, 0)] = feats

        return jnp.where(col == im, BIG, d2m)

    jax.lax.fori_loop(0, K_NN + 1, step, d2)


def _knn_feats_call(coords):
    B, N, _ = coords.shape
    T = min(N, 128)
    return pl.pallas_call(
        partial(_knn_feats_body, T=T, N=N),
        grid=(B, N // T),
        in_specs=[pl.BlockSpec((1, N, 3), lambda b, t: (b, 0, 0))],
        out_specs=pl.BlockSpec((1, K_NN, T, 10), lambda b, t: (b, 0, t, 0)),
        out_shape=jax.ShapeDtypeStruct((B, K_NN, N, 10), jnp.float32),
    )(coords)


def _relu(v):
    return jnp.maximum(v, 0.0)


def _lrelu02(v):
    return jnp.where(v > 0, v, 0.2 * v)


def _smlp(a, w, b, g, be, act):
    return act((_bmm(a, w[...]) + b[...]) * g[...] + be[...])


def _attpool(cat, w, b, K, T, C):
    attn = (_bmm(cat.reshape(K * T, C), w[...]) + b[...]).reshape(K, T, C)
    attn = attn - jnp.max(attn, axis=0, keepdims=True)
    attn = jnp.exp(attn)
    attn = attn / jnp.sum(attn, axis=0, keepdims=True)
    return jnp.sum(attn * cat, axis=0)


def _lfa_dense_body(x_ref, f_ref, w1, b1, g1, e1, wl1, bl1, gl1, el1, wp1, bp1,
                    wmp, bmp, gmp, emp, wl2, bl2, gl2, el2, wp2, bp2,
                    w2, b2, g2, e2, wr, br, gr, er, out_ref, *, T, dout):
    h, qd = dout // 2, dout // 4
    x = x_ref[0]
    ff = f_ref[0].reshape(K_NN * T, 10)

    x1 = _smlp(x, w1, b1, g1, e1, _lrelu02)
    sf1 = _smlp(ff, wl1, bl1, gl1, el1, _relu).reshape(K_NN, T, h)
    cat1 = jnp.concatenate(
        [sf1, jnp.broadcast_to(x1[None], (K_NN, T, h))], axis=2)
    pooled = _attpool(cat1, wp1, bp1, K_NN, T, dout)

    x2 = _smlp(pooled, wmp, bmp, gmp, emp, _relu)
    sf2 = _smlp(ff, wl2, bl2, gl2, el2, _relu).reshape(K_NN, T, qd)
    cat2 = jnp.concatenate(
        [sf2, jnp.broadcast_to(x2[None], (K_NN, T, qd))], axis=2)
    pooled2 = _attpool(cat2, wp2, bp2, K_NN, T, h)

    xm = _smlp(pooled2, w2, b2, g2, e2, _relu)
    res = _smlp(x, wr, br, gr, er, _relu)
    out = jnp.concatenate([xm, res], axis=1)
    out_ref[0] = jnp.where(out > 0, out, 0.01 * out)


def _mlp_t(p):
    return (p["W"].T, p["b"][None, :], p["g"][None, :], p["be"][None, :])


def _lin_t(p):
    return p["W"].T, p["b"][None, :]


def _lfa_dense_call(x, feats, p, dout):
    B, N, din = x.shape
    T = min(N, 512 if dout <= 128 else 128)
    ws = [*_mlp_t(p["mlp1"]), *_mlp_t(p["lse1"]), *_lin_t(p["pool1"]),
          *_mlp_t(p["mlp_pool1"]), *_mlp_t(p["lse2"]), *_lin_t(p["pool2"]),
          *_mlp_t(p["mlp2"]), *_mlp_t(p["residual"])]
    wspecs = [pl.BlockSpec(w.shape, lambda b, t: (0, 0)) for w in ws]
    return pl.pallas_call(
        partial(_lfa_dense_body, T=T, dout=dout),
        grid=(B, N // T),
        in_specs=[pl.BlockSpec((1, T, din), lambda b, t: (b, t, 0)),
                  pl.BlockSpec((1, K_NN, T, 10), lambda b, t: (b, 0, t, 0)),
                  *wspecs],
        out_specs=pl.BlockSpec((1, T, 2 * dout), lambda b, t: (b, t, 0)),
        out_shape=jax.ShapeDtypeStruct((B, N, 2 * dout), jnp.float32),
    )(x, feats, *ws)


def _chain_body(*refs, n_in, layers, acts):
    in_refs = refs[:n_in]
    w_refs = refs[n_in:-1]
    out_ref = refs[-1]
    hcat = jnp.concatenate([r[0] for r in in_refs], axis=1)
    for i in range(layers):
        w, b, g, be = w_refs[4 * i:4 * i + 4]
        act = _relu if acts[i] == "relu" else _lrelu02
        hcat = _smlp(hcat, w, b, g, be, act)
    out_ref[0] = hcat


def _chain_call(xs, wbs, acts):
    B, N = xs[0].shape[0], xs[0].shape[1]
    T = min(N, 512)
    cout = wbs[-1][0].shape[1]
    flat_ws = [a for wb in wbs for a in wb]
    in_specs = [pl.BlockSpec((1, T, x.shape[2]), lambda b, t: (b, t, 0))
                for x in xs]
    in_specs += [pl.BlockSpec(w.shape, lambda b, t: (0, 0)) for w in flat_ws]
    return pl.pallas_call(
        partial(_chain_body, n_in=len(xs), layers=len(wbs), acts=acts),
        grid=(B, N // T),
        in_specs=in_specs,
        out_specs=pl.BlockSpec((1, T, cout), lambda b, t: (b, t, 0)),
        out_shape=jax.ShapeDtypeStruct((B, N, cout), jnp.float32),
    )(*xs, *flat_ws)


def _interp_ids(S, T):
    return (np.arange(T) * S) // T


def kernel(input, params):
    B, N, _ = input.shape
    coords = input[..., :3]

    fcs = (params["fc_start"]["W"].T, params["fc_start"]["b"][None, :],
           params["bn_start"]["g"][None, :], params["bn_start"]["be"][None, :])
    x = _chain_call([input], [fcs], ["lrelu02"])

    douts = [16, 64, 128, 256]
    x_stack, np_stack = [], []
    decim, target_N = 1, N
    for i in range(4):
        x = _lfa_dense_call(x, _knn_feats_call(coords), params[f"enc{i}"],
                            douts[i])
        x_stack.append(x)
        np_stack.append(target_N)
        target_N = max(1, N // (decim * 4))
        perm = jax.random.permutation(jax.random.key(100 + i),
                                      coords.shape[1])[:target_N]
        coords = coords[:, perm, :]
        x = x[:, perm, :]
        decim *= 4

    for i in range(4):
        up = np_stack.pop()
        x = jnp.take(x, _interp_ids(x.shape[1], up), axis=1)
        skip = x_stack.pop()
        if skip.shape[1] != target_N:
            skip = jnp.take(skip, _interp_ids(skip.shape[1], up), axis=1)
        x = _chain_call([x, skip], [_mlp_t(params[f"dec{i}"])], ["relu"])

    x = _chain_call([x], [_mlp_t(params["fc_end0"]),
                          _mlp_t(params["fc_end1"]),
                          _mlp_t(params["fc_end2"])],
                    ["relu", "relu", "relu"])
    return jnp.transpose(x, (0, 2, 1))


# speedup vs baseline: 8.5913x; 3.5050x over previous
import functools
from functools import partial

import jax
import jax.numpy as jnp
import numpy as np
from jax import lax
from jax.experimental import pallas as pl
from jax.experimental.pallas import tpu as pltpu
from jax.experimental.pallas import tpu_sc as plsc

K_NN = 16
BIG = 3.0e38

_DOT = partial(jax.lax.dot_general, precision=jax.lax.Precision.HIGHEST,
               preferred_element_type=jnp.float32)


def _bmm(a, b):
    return jax.lax.dot_general(a.astype(jnp.bfloat16), b.astype(jnp.bfloat16),
                               (((1,), (0,)), ((), ())),
                               preferred_element_type=jnp.float32)


def _d2_body(c_ref, out_ref, *, T, N):
    t = pl.program_id(1)
    c = c_ref[0]
    q = c_ref[0, pl.ds(t * T, T), :]
    cross = jax.lax.dot_general(q.astype(jnp.bfloat16), c.astype(jnp.bfloat16),
                                (((1,), (1,)), ((), ())),
                                preferred_element_type=jnp.float32)
    q0, q1, q2 = q[:, 0:1], q[:, 1:2], q[:, 2:3]
    n2q = (q0 * q0 + q1 * q1) + q2 * q2
    n2c = _DOT(jnp.ones((1, 3), jnp.float32), c * c,
               (((1,), (1,)), ((), ())))
    out_ref[0] = (n2q + n2c) - 2.0 * cross


def _d2_call(coords):
    B, N, _ = coords.shape
    T = min(N, 512)
    return pl.pallas_call(
        partial(_d2_body, T=T, N=N),
        grid=(B, N // T),
        in_specs=[pl.BlockSpec((1, N, 3), lambda b, t: (b, 0, 0))],
        out_specs=pl.BlockSpec((1, T, N), lambda b, t: (b, t, 0)),
        out_shape=jax.ShapeDtypeStruct((B, N, N), jnp.float32),
    )(coords)


CH = 32


@functools.lru_cache(None)
def _sc_knn_make(B, N):
    NC = N // CH
    G = N // 16
    NW = 32
    mesh = plsc.VectorSubcoreMesh(core_axis_name="c", subcore_axis_name="s")

    @functools.partial(
        pl.kernel, mesh=mesh,
        compiler_params=pltpu.CompilerParams(needs_layout_passes=False),
        out_type=jax.ShapeDtypeStruct((B, K_NN, N * 16), jnp.float32),
        scratch_types=[
            pltpu.VMEM((16 * N,), jnp.float32),
            pltpu.VMEM((NC * 16,), jnp.float32),
            pltpu.VMEM((K_NN * 16 * 16,), jnp.float32),
            pltpu.VMEM((N,), jnp.float32),
            pltpu.VMEM((N,), jnp.float32),
            pltpu.VMEM((N,), jnp.float32),
        ],
    )
    def knn(d2_hbm, cx_hbm, cy_hbm, cz_hbm, out_hbm, dt, M, ob, tx, ty, tz):
        wid = lax.axis_index("s") * 2 + lax.axis_index("c")
        lanes = lax.broadcasted_iota(jnp.int32, (16,), 0)
        lanesN = lanes * N
        bigv = jnp.full((16,), BIG, jnp.float32)

        def gat(j):
            return plsc.load_gather(dt, [lanesN + j])

        for b in range(B):
            pltpu.sync_copy(cx_hbm.at[b], tx)
            pltpu.sync_copy(cy_hbm.at[b], ty)
            pltpu.sync_copy(cz_hbm.at[b], tz)
            trip = (G - wid + NW - 1) // NW

            def group_body(gi, _, b=b):
                n0 = (wid + gi * NW) * 16
                for r in range(16):
                    pltpu.sync_copy(d2_hbm.at[b, n0 + r, :],
                                    dt.at[pl.ds(r * N, N)])
                qx = tx[pl.ds(n0, 16)]
                qy = ty[pl.ds(n0, 16)]
                qz = tz[pl.ds(n0, 16)]

                def build(c, _):
                    j0 = jnp.full((16,), c * CH, jnp.int32)
                    m = gat(j0)
                    for p in range(1, CH):
                        m = jnp.minimum(m, gat(j0 + p))
                    M[pl.ds(c * 16, 16)] = m
                    return 0

                lax.fori_loop(0, NC, build, 0)

                def extract(k, _):
                    mval = M[pl.ds(0, 16)]
                    midx = jnp.zeros((16,), jnp.int32)
                    for c in range(1, NC):
                        v = M[pl.ds(c * 16, 16)]
                        msk = v < mval
                        mval = jnp.where(msk, v, mval)
                        midx = jnp.where(msk, c, midx)
                    base = midx * CH
                    cur = jnp.full((16,), jnp.inf, jnp.float32)
                    m2 = jnp.full((16,), jnp.inf, jnp.float32)
                    jbest = jnp.zeros((16,), jnp.int32)
                    for p in range(CH):
                        jv = base + p
                        v = gat(jv)
                        mk = v < cur
                        m2 = jnp.where(mk, cur, jnp.where(v < m2, v, m2))
                        cur = jnp.where(mk, v, cur)
                        jbest = jnp.where(mk, jv, jbest)
                    plsc.store_scatter(dt, [lanesN + jbest], bigv)
                    plsc.store_scatter(M, [midx * 16 + lanes], m2)
                    nbx = plsc.load_gather(tx, [jbest])
                    nby = plsc.load_gather(ty, [jbest])
                    nbz = plsc.load_gather(tz, [jbest])
                    dist = jnp.maximum(cur, 0.0)
                    kf = jnp.full((16,), jnp.maximum(k - 1, 0), jnp.int32)
                    zv = jnp.zeros((16,), jnp.float32)
                    vals = [qx, qy, qz, nbx, nby, nbz, qx - nbx, qy - nby,
                            qz - nbz, dist, zv, zv, zv, zv, zv, zv]
                    for ch, val in enumerate(vals):
                        plsc.store_scatter(
                            ob, [kf * 256 + lanes * 16 + ch], val)
                    return 0

                lax.fori_loop(0, K_NN + 1, extract, 0)
                for k in range(K_NN):
                    pltpu.sync_copy(ob.at[pl.ds(k * 256, 256)],
                                    out_hbm.at[b, k, pl.ds(n0 * 16, 256)])
                return 0

            lax.fori_loop(0, trip, group_body, 0)

    return knn


def _knn_feats_tc_body(c_ref, out_ref, *, T, N):
    t = pl.program_id(1)
    c = c_ref[0]
    q = c_ref[0, pl.ds(t * T, T), :]
    cross = jax.lax.dot_general(q.astype(jnp.bfloat16), c.astype(jnp.bfloat16),
                                (((1,), (1,)), ((), ())),
                                preferred_element_type=jnp.float32)
    q0, q1, q2 = q[:, 0:1], q[:, 1:2], q[:, 2:3]
    n2q = (q0 * q0 + q1 * q1) + q2 * q2
    n2c = _DOT(jnp.ones((1, 3), jnp.float32), c * c, (((1,), (1,)), ((), ())))
    d2 = (n2q + n2c) - 2.0 * cross
    col = jax.lax.broadcasted_iota(jnp.int32, (T, N), 1)
    zpad = jnp.zeros((T, 6), jnp.float32)

    def step(j, d2m):
        m = jnp.min(d2m, axis=1, keepdims=True)
        idxc = jnp.where(d2m == m, col, N)
        im = jnp.min(idxc, axis=1, keepdims=True)
        oh = (col == im).astype(jnp.float32)
        nb = _DOT(oh, c, (((1,), (0,)), ((), ())))
        dist = jnp.maximum(m, 0.0)
        feats = jnp.concatenate([q, nb, q - nb, dist, zpad], axis=1)

        @pl.when(j > 0)
        def _():
            out_ref[0, jnp.maximum(j - 1, 0)] = feats

        return jnp.where(col == im, BIG, d2m)

    jax.lax.fori_loop(0, K_NN + 1, step, d2)


def _knn_feats_call(coords):
    B, N, _ = coords.shape
    if N < 128:
        T = min(N, 128)
        return pl.pallas_call(
            partial(_knn_feats_tc_body, T=T, N=N),
            grid=(B, N // T),
            in_specs=[pl.BlockSpec((1, N, 3), lambda b, t: (b, 0, 0))],
            out_specs=pl.BlockSpec((1, K_NN, T, 16),
                                   lambda b, t: (b, 0, t, 0)),
            out_shape=jax.ShapeDtypeStruct((B, K_NN, N, 16), jnp.float32),
        )(coords)
    d2 = _d2_call(coords)
    f = _sc_knn_make(B, N)(d2, coords[..., 0], coords[..., 1], coords[..., 2])
    return f.reshape(B, K_NN, N, 16)


def _relu(v):
    return jnp.maximum(v, 0.0)


def _lrelu02(v):
    return jnp.where(v > 0, v, 0.2 * v)


def _smlp(a, w, b, g, be, act):
    return act((_bmm(a, w[...]) + b[...]) * g[...] + be[...])


def _attpool(cat, w, b, K, T, C):
    attn = (_bmm(cat.reshape(K * T, C), w[...]) + b[...]).reshape(K, T, C)
    attn = attn - jnp.max(attn, axis=0, keepdims=True)
    attn = jnp.exp(attn)
    attn = attn / jnp.sum(attn, axis=0, keepdims=True)
    return jnp.sum(attn * cat, axis=0)


def _lfa_dense_body(x_ref, f_ref, w1, b1, g1, e1, wl1, bl1, gl1, el1, wp1, bp1,
                    wmp, bmp, gmp, emp, wl2, bl2, gl2, el2, wp2, bp2,
                    w2, b2, g2, e2, wr, br, gr, er, out_ref, *, T, dout):
    h, qd = dout // 2, dout // 4
    x = x_ref[0]
    ff = f_ref[0].reshape(K_NN * T, 16)

    x1 = _smlp(x, w1, b1, g1, e1, _lrelu02)
    sf1 = _smlp(ff, wl1, bl1, gl1, el1, _relu).reshape(K_NN, T, h)
    cat1 = jnp.concatenate(
        [sf1, jnp.broadcast_to(x1[None], (K_NN, T, h))], axis=2)
    pooled = _attpool(cat1, wp1, bp1, K_NN, T, dout)

    x2 = _smlp(pooled, wmp, bmp, gmp, emp, _relu)
    sf2 = _smlp(ff, wl2, bl2, gl2, el2, _relu).reshape(K_NN, T, qd)
    cat2 = jnp.concatenate(
        [sf2, jnp.broadcast_to(x2[None], (K_NN, T, qd))], axis=2)
    pooled2 = _attpool(cat2, wp2, bp2, K_NN, T, h)

    xm = _smlp(pooled2, w2, b2, g2, e2, _relu)
    res = _smlp(x, wr, br, gr, er, _relu)
    out = jnp.concatenate([xm, res], axis=1)
    out_ref[0] = jnp.where(out > 0, out, 0.01 * out)


def _mlp_t(p):
    return (p["W"].T, p["b"][None, :], p["g"][None, :], p["be"][None, :])


def _lin_t(p):
    return p["W"].T, p["b"][None, :]


def _lfa_dense_call(x, feats, p, dout):
    B, N, din = x.shape
    T = min(N, 512 if dout <= 128 else 128)
    def _pad16(wb):
        w, b, g, be = wb
        return (jnp.concatenate([w, jnp.zeros((6, w.shape[1]), w.dtype)], 0),
                b, g, be)

    ws = [*_mlp_t(p["mlp1"]), *_pad16(_mlp_t(p["lse1"])), *_lin_t(p["pool1"]),
          *_mlp_t(p["mlp_pool1"]), *_pad16(_mlp_t(p["lse2"])),
          *_lin_t(p["pool2"]), *_mlp_t(p["mlp2"]), *_mlp_t(p["residual"])]
    wspecs = [pl.BlockSpec(w.shape, lambda b, t: (0, 0)) for w in ws]
    return pl.pallas_call(
        partial(_lfa_dense_body, T=T, dout=dout),
        grid=(B, N // T),
        in_specs=[pl.BlockSpec((1, T, din), lambda b, t: (b, t, 0)),
                  pl.BlockSpec((1, K_NN, T, 16), lambda b, t: (b, 0, t, 0)),
                  *wspecs],
        out_specs=pl.BlockSpec((1, T, 2 * dout), lambda b, t: (b, t, 0)),
        out_shape=jax.ShapeDtypeStruct((B, N, 2 * dout), jnp.float32),
    )(x, feats, *ws)


def _chain_body(*refs, n_in, layers, acts):
    in_refs = refs[:n_in]
    w_refs = refs[n_in:-1]
    out_ref = refs[-1]
    hcat = jnp.concatenate([r[0] for r in in_refs], axis=1)
    for i in range(layers):
        w, b, g, be = w_refs[4 * i:4 * i + 4]
        act = _relu if acts[i] == "relu" else _lrelu02
        hcat = _smlp(hcat, w, b, g, be, act)
    out_ref[0] = hcat


def _chain_call(xs, wbs, acts):
    B, N = xs[0].shape[0], xs[0].shape[1]
    T = min(N, 512)
    cout = wbs[-1][0].shape[1]
    flat_ws = [a for wb in wbs for a in wb]
    in_specs = [pl.BlockSpec((1, T, x.shape[2]), lambda b, t: (b, t, 0))
                for x in xs]
    in_specs += [pl.BlockSpec(w.shape, lambda b, t: (0, 0)) for w in flat_ws]
    return pl.pallas_call(
        partial(_chain_body, n_in=len(xs), layers=len(wbs), acts=acts),
        grid=(B, N // T),
        in_specs=in_specs,
        out_specs=pl.BlockSpec((1, T, cout), lambda b, t: (b, t, 0)),
        out_shape=jax.ShapeDtypeStruct((B, N, cout), jnp.float32),
    )(*xs, *flat_ws)


def _interp_ids(S, T):
    return (np.arange(T) * S) // T


def kernel(input, params):
    B, N, _ = input.shape
    coords = input[..., :3]

    fcs = (params["fc_start"]["W"].T, params["fc_start"]["b"][None, :],
           params["bn_start"]["g"][None, :], params["bn_start"]["be"][None, :])
    x = _chain_call([input], [fcs], ["lrelu02"])

    douts = [16, 64, 128, 256]
    x_stack, np_stack = [], []
    decim, target_N = 1, N
    for i in range(4):
        x = _lfa_dense_call(x, _knn_feats_call(coords), params[f"enc{i}"],
                            douts[i])
        x_stack.append(x)
        np_stack.append(target_N)
        target_N = max(1, N // (decim * 4))
        perm = jax.random.permutation(jax.random.key(100 + i),
                                      coords.shape[1])[:target_N]
        coords = coords[:, perm, :]
        x = x[:, perm, :]
        decim *= 4

    for i in range(4):
        up = np_stack.pop()
        x = jnp.take(x, _interp_ids(x.shape[1], up), axis=1)
        skip = x_stack.pop()
        if skip.shape[1] != target_N:
            skip = jnp.take(skip, _interp_ids(skip.shape[1], up), axis=1)
        x = _chain_call([x, skip], [_mlp_t(params[f"dec{i}"])], ["relu"])

    x = _chain_call([x], [_mlp_t(params["fc_end0"]),
                          _mlp_t(params["fc_end1"]),
                          _mlp_t(params["fc_end2"])],
                    ["relu", "relu", "relu"])
    return jnp.transpose(x, (0, 2, 1))


# speedup vs baseline: 8.6814x; 1.0105x over previous
import functools
from functools import partial

import jax
import jax.numpy as jnp
import numpy as np
from jax import lax
from jax.experimental import pallas as pl
from jax.experimental.pallas import tpu as pltpu
from jax.experimental.pallas import tpu_sc as plsc

K_NN = 16
BIG = 3.0e38

_DOT = partial(jax.lax.dot_general, precision=jax.lax.Precision.HIGHEST,
               preferred_element_type=jnp.float32)


def _bmm(a, b):
    return jax.lax.dot_general(a.astype(jnp.bfloat16), b.astype(jnp.bfloat16),
                               (((1,), (0,)), ((), ())),
                               preferred_element_type=jnp.float32)


def _d2_body(c_ref, out_ref, *, T, N):
    t = pl.program_id(1)
    c = c_ref[0]
    q = c_ref[0, pl.ds(t * T, T), :]
    cross = jax.lax.dot_general(q.astype(jnp.bfloat16), c.astype(jnp.bfloat16),
                                (((1,), (1,)), ((), ())),
                                preferred_element_type=jnp.float32)
    q0, q1, q2 = q[:, 0:1], q[:, 1:2], q[:, 2:3]
    n2q = (q0 * q0 + q1 * q1) + q2 * q2
    n2c = _DOT(jnp.ones((1, 3), jnp.float32), c * c,
               (((1,), (1,)), ((), ())))
    out_ref[0] = (n2q + n2c) - 2.0 * cross


def _d2_call(coords):
    B, N, _ = coords.shape
    T = min(N, 512)
    return pl.pallas_call(
        partial(_d2_body, T=T, N=N),
        grid=(B, N // T),
        in_specs=[pl.BlockSpec((1, N, 3), lambda b, t: (b, 0, 0))],
        out_specs=pl.BlockSpec((1, T, N), lambda b, t: (b, t, 0)),
        out_shape=jax.ShapeDtypeStruct((B, N, N), jnp.float32),
    )(coords)


CH = 32


@functools.lru_cache(None)
def _sc_knn_make(B, N):
    NC = N // CH
    G = N // 16
    NW = 32
    mesh = plsc.VectorSubcoreMesh(core_axis_name="c", subcore_axis_name="s")

    @functools.partial(
        pl.kernel, mesh=mesh,
        compiler_params=pltpu.CompilerParams(needs_layout_passes=False),
        out_type=jax.ShapeDtypeStruct((B, G, K_NN * 256), jnp.float32),
        scratch_types=[
            pltpu.VMEM((16 * N,), jnp.float32),
            pltpu.VMEM((NC * 16,), jnp.float32),
            pltpu.VMEM((K_NN * 16 * 16,), jnp.float32),
            pltpu.VMEM((N,), jnp.float32),
            pltpu.VMEM((N,), jnp.float32),
            pltpu.VMEM((N,), jnp.float32),
        ],
    )
    def knn(d2_hbm, cx_hbm, cy_hbm, cz_hbm, out_hbm, dt, M, ob, tx, ty, tz):
        wid = lax.axis_index("s") * 2 + lax.axis_index("c")
        lanes = lax.broadcasted_iota(jnp.int32, (16,), 0)
        lanesN = lanes * N
        bigv = jnp.full((16,), BIG, jnp.float32)

        def gat(j):
            return plsc.load_gather(dt, [lanesN + j])

        for b in range(B):
            pltpu.sync_copy(cx_hbm.at[b], tx)
            pltpu.sync_copy(cy_hbm.at[b], ty)
            pltpu.sync_copy(cz_hbm.at[b], tz)
            trip = (G - wid + NW - 1) // NW

            def group_body(gi, _, b=b):
                g = wid + gi * NW
                n0 = g * 16
                pltpu.sync_copy(d2_hbm.at[b, pl.ds(n0 * N, 16 * N)], dt)
                qx = tx[pl.ds(n0, 16)]
                qy = ty[pl.ds(n0, 16)]
                qz = tz[pl.ds(n0, 16)]

                def build(c, _):
                    j0 = jnp.full((16,), c * CH, jnp.int32)
                    vs = [gat(j0 + p) for p in range(CH)]
                    while len(vs) > 1:
                        vs = [jnp.minimum(vs[i], vs[i + 1])
                              for i in range(0, len(vs), 2)]
                    M[pl.ds(c * 16, 16)] = vs[0]
                    return 0

                lax.fori_loop(0, NC, build, 0)

                def extract(k, _):
                    nodes = [(M[pl.ds(c * 16, 16)],
                              jnp.full((16,), c, jnp.int32))
                             for c in range(NC)]
                    while len(nodes) > 1:
                        nxt = []
                        for i in range(0, len(nodes), 2):
                            (va, ia), (vb, ib) = nodes[i], nodes[i + 1]
                            mk = vb < va
                            nxt.append((jnp.where(mk, vb, va),
                                        jnp.where(mk, ib, ia)))
                        nodes = nxt
                    mval, midx = nodes[0]
                    base = midx * CH
                    tri = [(gat(base + p), base + p,
                            jnp.full((16,), jnp.inf, jnp.float32))
                           for p in range(CH)]
                    while len(tri) > 1:
                        nxt = []
                        for i in range(0, len(tri), 2):
                            (va, ja, sa), (vb, jb, sb) = tri[i], tri[i + 1]
                            mk = vb < va
                            nxt.append((jnp.where(mk, vb, va),
                                        jnp.where(mk, jb, ja),
                                        jnp.minimum(jnp.minimum(sa, sb),
                                                    jnp.where(mk, va, vb))))
                        tri = nxt
                    cur, jbest, m2 = tri[0]
                    plsc.store_scatter(dt, [lanesN + jbest], bigv)
                    plsc.store_scatter(M, [midx * 16 + lanes], m2)
                    nbx = plsc.load_gather(tx, [jbest])
                    nby = plsc.load_gather(ty, [jbest])
                    nbz = plsc.load_gather(tz, [jbest])
                    dist = jnp.maximum(cur, 0.0)
                    kf = jnp.full((16,), jnp.maximum(k - 1, 0), jnp.int32)
                    zv = jnp.zeros((16,), jnp.float32)
                    vals = [qx, qy, qz, nbx, nby, nbz, qx - nbx, qy - nby,
                            qz - nbz, dist, zv, zv, zv, zv, zv, zv]
                    for ch, val in enumerate(vals):
                        plsc.store_scatter(
                            ob, [kf * 256 + lanes * 16 + ch], val)
                    return 0

                lax.fori_loop(0, K_NN + 1, extract, 0)
                pltpu.sync_copy(ob, out_hbm.at[b, g])
                return 0

            lax.fori_loop(0, trip, group_body, 0)

    return knn


def _knn_feats_tc_body(c_ref, out_ref, *, T, N):
    t = pl.program_id(1)
    c = c_ref[0]
    q = c_ref[0, pl.ds(t * T, T), :]
    cross = jax.lax.dot_general(q.astype(jnp.bfloat16), c.astype(jnp.bfloat16),
                                (((1,), (1,)), ((), ())),
                                preferred_element_type=jnp.float32)
    q0, q1, q2 = q[:, 0:1], q[:, 1:2], q[:, 2:3]
    n2q = (q0 * q0 + q1 * q1) + q2 * q2
    n2c = _DOT(jnp.ones((1, 3), jnp.float32), c * c, (((1,), (1,)), ((), ())))
    d2 = (n2q + n2c) - 2.0 * cross
    col = jax.lax.broadcasted_iota(jnp.int32, (T, N), 1)
    zpad = jnp.zeros((T, 6), jnp.float32)

    def step(j, d2m):
        m = jnp.min(d2m, axis=1, keepdims=True)
        idxc = jnp.where(d2m == m, col, N)
        im = jnp.min(idxc, axis=1, keepdims=True)
        oh = (col == im).astype(jnp.float32)
        nb = _DOT(oh, c, (((1,), (0,)), ((), ())))
        dist = jnp.maximum(m, 0.0)
        feats = jnp.concatenate([q, nb, q - nb, dist, zpad], axis=1)

        @pl.when(j > 0)
        def _():
            out_ref[0, jnp.maximum(j - 1, 0)] = feats

        return jnp.where(col == im, BIG, d2m)

    jax.lax.fori_loop(0, K_NN + 1, step, d2)


def _knn_feats_call(coords):
    B, N, _ = coords.shape
    if N < 128:
        T = min(N, 128)
        return pl.pallas_call(
            partial(_knn_feats_tc_body, T=T, N=N),
            grid=(B, N // T),
            in_specs=[pl.BlockSpec((1, N, 3), lambda b, t: (b, 0, 0))],
            out_specs=pl.BlockSpec((1, K_NN, T, 16),
                                   lambda b, t: (b, 0, t, 0)),
            out_shape=jax.ShapeDtypeStruct((B, K_NN, N, 16), jnp.float32),
        )(coords)
    d2 = _d2_call(coords).reshape(B, N * N)
    f = _sc_knn_make(B, N)(d2, coords[..., 0], coords[..., 1], coords[..., 2])
    return f.reshape(B, N // 16, K_NN, 16, 16).transpose(0, 2, 1, 3, 4
                                                         ).reshape(B, K_NN, N, 16)


def _relu(v):
    return jnp.maximum(v, 0.0)


def _lrelu02(v):
    return jnp.where(v > 0, v, 0.2 * v)


def _smlp(a, w, b, g, be, act):
    return act((_bmm(a, w[...]) + b[...]) * g[...] + be[...])


def _attpool(cat, w, b, K, T, C):
    attn = (_bmm(cat.reshape(K * T, C), w[...]) + b[...]).reshape(K, T, C)
    attn = attn - jnp.max(attn, axis=0, keepdims=True)
    attn = jnp.exp(attn)
    attn = attn / jnp.sum(attn, axis=0, keepdims=True)
    return jnp.sum(attn * cat, axis=0)


def _lfa_dense_body(x_ref, f_ref, w1, b1, g1, e1, wl1, bl1, gl1, el1, wp1, bp1,
                    wmp, bmp, gmp, emp, wl2, bl2, gl2, el2, wp2, bp2,
                    w2, b2, g2, e2, wr, br, gr, er, out_ref, *, T, dout):
    h, qd = dout // 2, dout // 4
    x = x_ref[0]
    ff = f_ref[0].reshape(K_NN * T, 16)

    x1 = _smlp(x, w1, b1, g1, e1, _lrelu02)
    sf1 = _smlp(ff, wl1, bl1, gl1, el1, _relu).reshape(K_NN, T, h)
    cat1 = jnp.concatenate(
        [sf1, jnp.broadcast_to(x1[None], (K_NN, T, h))], axis=2)
    pooled = _attpool(cat1, wp1, bp1, K_NN, T, dout)

    x2 = _smlp(pooled, wmp, bmp, gmp, emp, _relu)
    sf2 = _smlp(ff, wl2, bl2, gl2, el2, _relu).reshape(K_NN, T, qd)
    cat2 = jnp.concatenate(
        [sf2, jnp.broadcast_to(x2[None], (K_NN, T, qd))], axis=2)
    pooled2 = _attpool(cat2, wp2, bp2, K_NN, T, h)

    xm = _smlp(pooled2, w2, b2, g2, e2, _relu)
    res = _smlp(x, wr, br, gr, er, _relu)
    out = jnp.concatenate([xm, res], axis=1)
    out_ref[0] = jnp.where(out > 0, out, 0.01 * out)


def _mlp_t(p):
    return (p["W"].T, p["b"][None, :], p["g"][None, :], p["be"][None, :])


def _lin_t(p):
    return p["W"].T, p["b"][None, :]


def _lfa_dense_call(x, feats, p, dout):
    B, N, din = x.shape
    T = min(N, 512 if dout <= 128 else 128)
    def _pad16(wb):
        w, b, g, be = wb
        return (jnp.concatenate([w, jnp.zeros((6, w.shape[1]), w.dtype)], 0),
                b, g, be)

    ws = [*_mlp_t(p["mlp1"]), *_pad16(_mlp_t(p["lse1"])), *_lin_t(p["pool1"]),
          *_mlp_t(p["mlp_pool1"]), *_pad16(_mlp_t(p["lse2"])),
          *_lin_t(p["pool2"]), *_mlp_t(p["mlp2"]), *_mlp_t(p["residual"])]
    wspecs = [pl.BlockSpec(w.shape, lambda b, t: (0, 0)) for w in ws]
    return pl.pallas_call(
        partial(_lfa_dense_body, T=T, dout=dout),
        grid=(B, N // T),
        in_specs=[pl.BlockSpec((1, T, din), lambda b, t: (b, t, 0)),
                  pl.BlockSpec((1, K_NN, T, 16), lambda b, t: (b, 0, t, 0)),
                  *wspecs],
        out_specs=pl.BlockSpec((1, T, 2 * dout), lambda b, t: (b, t, 0)),
        out_shape=jax.ShapeDtypeStruct((B, N, 2 * dout), jnp.float32),
    )(x, feats, *ws)


def _chain_body(*refs, n_in, layers, acts):
    in_refs = refs[:n_in]
    w_refs = refs[n_in:-1]
    out_ref = refs[-1]
    hcat = jnp.concatenate([r[0] for r in in_refs], axis=1)
    for i in range(layers):
        w, b, g, be = w_refs[4 * i:4 * i + 4]
        act = _relu if acts[i] == "relu" else _lrelu02
        hcat = _smlp(hcat, w, b, g, be, act)
    out_ref[0] = hcat


def _chain_call(xs, wbs, acts):
    B, N = xs[0].shape[0], xs[0].shape[1]
    T = min(N, 512)
    cout = wbs[-1][0].shape[1]
    flat_ws = [a for wb in wbs for a in wb]
    in_specs = [pl.BlockSpec((1, T, x.shape[2]), lambda b, t: (b, t, 0))
                for x in xs]
    in_specs += [pl.BlockSpec(w.shape, lambda b, t: (0, 0)) for w in flat_ws]
    return pl.pallas_call(
        partial(_chain_body, n_in=len(xs), layers=len(wbs), acts=acts),
        grid=(B, N // T),
        in_specs=in_specs,
        out_specs=pl.BlockSpec((1, T, cout), lambda b, t: (b, t, 0)),
        out_shape=jax.ShapeDtypeStruct((B, N, cout), jnp.float32),
    )(*xs, *flat_ws)


def _interp_ids(S, T):
    return (np.arange(T) * S) // T


def kernel(input, params):
    B, N, _ = input.shape
    coords = input[..., :3]

    fcs = (params["fc_start"]["W"].T, params["fc_start"]["b"][None, :],
           params["bn_start"]["g"][None, :], params["bn_start"]["be"][None, :])
    x = _chain_call([input], [fcs], ["lrelu02"])

    douts = [16, 64, 128, 256]
    x_stack, np_stack = [], []
    decim, target_N = 1, N
    for i in range(4):
        x = _lfa_dense_call(x, _knn_feats_call(coords), params[f"enc{i}"],
                            douts[i])
        x_stack.append(x)
        np_stack.append(target_N)
        target_N = max(1, N // (decim * 4))
        perm = jax.random.permutation(jax.random.key(100 + i),
                                      coords.shape[1])[:target_N]
        coords = coords[:, perm, :]
        x = x[:, perm, :]
        decim *= 4

    for i in range(4):
        up = np_stack.pop()
        x = jnp.take(x, _interp_ids(x.shape[1], up), axis=1)
        skip = x_stack.pop()
        if skip.shape[1] != target_N:
            skip = jnp.take(skip, _interp_ids(skip.shape[1], up), axis=1)
        x = _chain_call([x, skip], [_mlp_t(params[f"dec{i}"])], ["relu"])

    x = _chain_call([x], [_mlp_t(params["fc_end0"]),
                          _mlp_t(params["fc_end1"]),
                          _mlp_t(params["fc_end2"])],
                    ["relu", "relu", "relu"])
    return jnp.transpose(x, (0, 2, 1))


# speedup vs baseline: 12.9934x; 1.4967x over previous
import functools
from functools import partial

import jax
import jax.numpy as jnp
import numpy as np
from jax import lax
from jax.experimental import pallas as pl
from jax.experimental.pallas import tpu as pltpu
from jax.experimental.pallas import tpu_sc as plsc

K_NN = 16
BIG = 3.0e38

_DOT = partial(jax.lax.dot_general, precision=jax.lax.Precision.HIGHEST,
               preferred_element_type=jnp.float32)


def _bmm(a, b):
    return jax.lax.dot_general(a.astype(jnp.bfloat16), b.astype(jnp.bfloat16),
                               (((1,), (0,)), ((), ())),
                               preferred_element_type=jnp.float32)


def _d2_body(c_ref, out_ref, *, T, N):
    t = pl.program_id(1)
    c = c_ref[0]
    q = c_ref[0, pl.ds(t * T, T), :]
    cross = jax.lax.dot_general(q.astype(jnp.bfloat16), c.astype(jnp.bfloat16),
                                (((1,), (1,)), ((), ())),
                                preferred_element_type=jnp.float32)
    q0, q1, q2 = q[:, 0:1], q[:, 1:2], q[:, 2:3]
    n2q = (q0 * q0 + q1 * q1) + q2 * q2
    n2c = _DOT(jnp.ones((1, 3), jnp.float32), c * c,
               (((1,), (1,)), ((), ())))
    out_ref[0] = (n2q + n2c) - 2.0 * cross


def _d2_call(coords):
    B, N, _ = coords.shape
    T = min(N, 512)
    return pl.pallas_call(
        partial(_d2_body, T=T, N=N),
        grid=(B, N // T),
        in_specs=[pl.BlockSpec((1, N, 3), lambda b, t: (b, 0, 0))],
        out_specs=pl.BlockSpec((1, T, N), lambda b, t: (b, t, 0)),
        out_shape=jax.ShapeDtypeStruct((B, N, N), jnp.float32),
    )(coords)


CH = 32


@functools.lru_cache(None)
def _sc_knn_make(B, N):
    NC = N // CH
    G = N // 16
    NW = 32
    mesh = plsc.VectorSubcoreMesh(core_axis_name="c", subcore_axis_name="s")

    @functools.partial(
        pl.kernel, mesh=mesh,
        compiler_params=pltpu.CompilerParams(needs_layout_passes=False),
        out_type=jax.ShapeDtypeStruct((B, G, K_NN * 256), jnp.float32),
        scratch_types=[
            pltpu.VMEM((16 * N,), jnp.float32),
            pltpu.VMEM((NC * 16,), jnp.float32),
            pltpu.VMEM((K_NN * 16 * 16,), jnp.float32),
            pltpu.VMEM((N,), jnp.float32),
            pltpu.VMEM((N,), jnp.float32),
            pltpu.VMEM((N,), jnp.float32),
        ],
    )
    def knn(d2_hbm, cx_hbm, cy_hbm, cz_hbm, out_hbm, dt, M, ob, tx, ty, tz):
        wid = lax.axis_index("s") * 2 + lax.axis_index("c")
        lanes = lax.broadcasted_iota(jnp.int32, (16,), 0)
        lanesN = lanes * N
        bigv = jnp.full((16,), BIG, jnp.float32)

        def gat(j):
            return plsc.load_gather(dt, [lanesN + j])

        for b in range(B):
            pltpu.sync_copy(cx_hbm.at[b], tx)
            pltpu.sync_copy(cy_hbm.at[b], ty)
            pltpu.sync_copy(cz_hbm.at[b], tz)
            trip = (G - wid + NW - 1) // NW

            def group_body(gi, _, b=b):
                g = wid + gi * NW
                n0 = g * 16
                pltpu.sync_copy(d2_hbm.at[b, pl.ds(n0 * N, 16 * N)], dt)
                qx = tx[pl.ds(n0, 16)]
                qy = ty[pl.ds(n0, 16)]
                qz = tz[pl.ds(n0, 16)]

                def build(c, _):
                    j0 = jnp.full((16,), c * CH, jnp.int32)
                    vs = [gat(j0 + ((lanes + p) & (CH - 1)))
                          for p in range(CH)]
                    while len(vs) > 1:
                        vs = [jnp.minimum(vs[i], vs[i + 1])
                              for i in range(0, len(vs), 2)]
                    M[pl.ds(c * 16, 16)] = vs[0]
                    return 0

                lax.fori_loop(0, NC, build, 0)

                def extract(k, _):
                    nodes = [(M[pl.ds(c * 16, 16)],
                              jnp.full((16,), c, jnp.int32))
                             for c in range(NC)]
                    while len(nodes) > 1:
                        nxt = []
                        for i in range(0, len(nodes), 2):
                            (va, ia), (vb, ib) = nodes[i], nodes[i + 1]
                            mk = vb < va
                            nxt.append((jnp.where(mk, vb, va),
                                        jnp.where(mk, ib, ia)))
                        nodes = nxt
                    mval, midx = nodes[0]
                    base = midx * CH
                    tri = [(gat(base + p), base + p,
                            jnp.full((16,), jnp.inf, jnp.float32))
                           for p in range(CH)]
                    while len(tri) > 1:
                        nxt = []
                        for i in range(0, len(tri), 2):
                            (va, ja, sa), (vb, jb, sb) = tri[i], tri[i + 1]
                            mk = vb < va
                            nxt.append((jnp.where(mk, vb, va),
                                        jnp.where(mk, jb, ja),
                                        jnp.minimum(jnp.minimum(sa, sb),
                                                    jnp.where(mk, va, vb))))
                        tri = nxt
                    cur, jbest, m2 = tri[0]
                    plsc.store_scatter(dt, [lanesN + jbest], bigv)
                    plsc.store_scatter(M, [midx * 16 + lanes], m2)
                    nbx = plsc.load_gather(tx, [jbest])
                    nby = plsc.load_gather(ty, [jbest])
                    nbz = plsc.load_gather(tz, [jbest])
                    dist = jnp.maximum(cur, 0.0)
                    kf = jnp.full((16,), jnp.maximum(k - 1, 0), jnp.int32)
                    zv = jnp.zeros((16,), jnp.float32)
                    vals = [qx, qy, qz, nbx, nby, nbz, qx - nbx, qy - nby,
                            qz - nbz, dist, zv, zv, zv, zv, zv, zv]
                    for ch, val in enumerate(vals):
                        plsc.store_scatter(
                            ob, [kf * 256 + lanes * 16 + ch], val)
                    return 0

                lax.fori_loop(0, K_NN + 1, extract, 0)
                pltpu.sync_copy(ob, out_hbm.at[b, g])
                return 0

            lax.fori_loop(0, trip, group_body, 0)

    return knn


def _knn_feats_tc_body(c_ref, out_ref, *, T, N):
    t = pl.program_id(1)
    c = c_ref[0]
    q = c_ref[0, pl.ds(t * T, T), :]
    cross = jax.lax.dot_general(q.astype(jnp.bfloat16), c.astype(jnp.bfloat16),
                                (((1,), (1,)), ((), ())),
                                preferred_element_type=jnp.float32)
    q0, q1, q2 = q[:, 0:1], q[:, 1:2], q[:, 2:3]
    n2q = (q0 * q0 + q1 * q1) + q2 * q2
    n2c = _DOT(jnp.ones((1, 3), jnp.float32), c * c, (((1,), (1,)), ((), ())))
    d2 = (n2q + n2c) - 2.0 * cross
    col = jax.lax.broadcasted_iota(jnp.int32, (T, N), 1)
    zpad = jnp.zeros((T, 6), jnp.float32)

    def step(j, d2m):
        m = jnp.min(d2m, axis=1, keepdims=True)
        idxc = jnp.where(d2m == m, col, N)
        im = jnp.min(idxc, axis=1, keepdims=True)
        oh = (col == im).astype(jnp.float32)
        nb = _DOT(oh, c, (((1,), (0,)), ((), ())))
        dist = jnp.maximum(m, 0.0)
        feats = jnp.concatenate([q, nb, q - nb, dist, zpad], axis=1)

        @pl.when(j > 0)
        def _():
            out_ref[0, jnp.maximum(j - 1, 0)] = feats

        return jnp.where(col == im, BIG, d2m)

    jax.lax.fori_loop(0, K_NN + 1, step, d2)


def _knn_feats_call(coords):
    B, N, _ = coords.shape
    if N < 128:
        T = min(N, 128)
        return pl.pallas_call(
            partial(_knn_feats_tc_body, T=T, N=N),
            grid=(B, N // T),
            in_specs=[pl.BlockSpec((1, N, 3), lambda b, t: (b, 0, 0))],
            out_specs=pl.BlockSpec((1, K_NN, T, 16),
                                   lambda b, t: (b, 0, t, 0)),
            out_shape=jax.ShapeDtypeStruct((B, K_NN, N, 16), jnp.float32),
        )(coords)
    d2 = _d2_call(coords).reshape(B, N * N)
    f = _sc_knn_make(B, N)(d2, coords[..., 0], coords[..., 1], coords[..., 2])
    return f.reshape(B, N // 16, K_NN, 16, 16).transpose(0, 2, 1, 3, 4
                                                         ).reshape(B, K_NN, N, 16)


def _relu(v):
    return jnp.maximum(v, 0.0)


def _lrelu02(v):
    return jnp.where(v > 0, v, 0.2 * v)


def _smlp(a, w, b, g, be, act):
    return act((_bmm(a, w[...]) + b[...]) * g[...] + be[...])


def _attpool(cat, w, b, K, T, C):
    attn = (_bmm(cat.reshape(K * T, C), w[...]) + b[...]).reshape(K, T, C)
    attn = attn - jnp.max(attn, axis=0, keepdims=True)
    attn = jnp.exp(attn)
    attn = attn / jnp.sum(attn, axis=0, keepdims=True)
    return jnp.sum(attn * cat, axis=0)


def _lfa_dense_body(x_ref, f_ref, w1, b1, g1, e1, wl1, bl1, gl1, el1, wp1, bp1,
                    wmp, bmp, gmp, emp, wl2, bl2, gl2, el2, wp2, bp2,
                    w2, b2, g2, e2, wr, br, gr, er, out_ref, *, T, dout):
    h, qd = dout // 2, dout // 4
    x = x_ref[0]
    ff = f_ref[0].reshape(K_NN * T, 16)

    x1 = _smlp(x, w1, b1, g1, e1, _lrelu02)
    sf1 = _smlp(ff, wl1, bl1, gl1, el1, _relu).reshape(K_NN, T, h)
    cat1 = jnp.concatenate(
        [sf1, jnp.broadcast_to(x1[None], (K_NN, T, h))], axis=2)
    pooled = _attpool(cat1, wp1, bp1, K_NN, T, dout)

    x2 = _smlp(pooled, wmp, bmp, gmp, emp, _relu)
    sf2 = _smlp(ff, wl2, bl2, gl2, el2, _relu).reshape(K_NN, T, qd)
    cat2 = jnp.concatenate(
        [sf2, jnp.broadcast_to(x2[None], (K_NN, T, qd))], axis=2)
    pooled2 = _attpool(cat2, wp2, bp2, K_NN, T, h)

    xm = _smlp(pooled2, w2, b2, g2, e2, _relu)
    res = _smlp(x, wr, br, gr, er, _relu)
    out = jnp.concatenate([xm, res], axis=1)
    out_ref[0] = jnp.where(out > 0, out, 0.01 * out)


def _mlp_t(p):
    return (p["W"].T, p["b"][None, :], p["g"][None, :], p["be"][None, :])


def _lin_t(p):
    return p["W"].T, p["b"][None, :]


def _lfa_dense_call(x, feats, p, dout):
    B, N, din = x.shape
    T = min(N, 512 if dout <= 128 else 128)
    def _pad16(wb):
        w, b, g, be = wb
        return (jnp.concatenate([w, jnp.zeros((6, w.shape[1]), w.dtype)], 0),
                b, g, be)

    ws = [*_mlp_t(p["mlp1"]), *_pad16(_mlp_t(p["lse1"])), *_lin_t(p["pool1"]),
          *_mlp_t(p["mlp_pool1"]), *_pad16(_mlp_t(p["lse2"])),
          *_lin_t(p["pool2"]), *_mlp_t(p["mlp2"]), *_mlp_t(p["residual"])]
    wspecs = [pl.BlockSpec(w.shape, lambda b, t: (0, 0)) for w in ws]
    return pl.pallas_call(
        partial(_lfa_dense_body, T=T, dout=dout),
        grid=(B, N // T),
        in_specs=[pl.BlockSpec((1, T, din), lambda b, t: (b, t, 0)),
                  pl.BlockSpec((1, K_NN, T, 16), lambda b, t: (b, 0, t, 0)),
                  *wspecs],
        out_specs=pl.BlockSpec((1, T, 2 * dout), lambda b, t: (b, t, 0)),
        out_shape=jax.ShapeDtypeStruct((B, N, 2 * dout), jnp.float32),
    )(x, feats, *ws)


def _chain_body(*refs, n_in, layers, acts):
    in_refs = refs[:n_in]
    w_refs = refs[n_in:-1]
    out_ref = refs[-1]
    hcat = jnp.concatenate([r[0] for r in in_refs], axis=1)
    for i in range(layers):
        w, b, g, be = w_refs[4 * i:4 * i + 4]
        act = _relu if acts[i] == "relu" else _lrelu02
        hcat = _smlp(hcat, w, b, g, be, act)
    out_ref[0] = hcat


def _chain_call(xs, wbs, acts):
    B, N = xs[0].shape[0], xs[0].shape[1]
    T = min(N, 512)
    cout = wbs[-1][0].shape[1]
    flat_ws = [a for wb in wbs for a in wb]
    in_specs = [pl.BlockSpec((1, T, x.shape[2]), lambda b, t: (b, t, 0))
                for x in xs]
    in_specs += [pl.BlockSpec(w.shape, lambda b, t: (0, 0)) for w in flat_ws]
    return pl.pallas_call(
        partial(_chain_body, n_in=len(xs), layers=len(wbs), acts=acts),
        grid=(B, N // T),
        in_specs=in_specs,
        out_specs=pl.BlockSpec((1, T, cout), lambda b, t: (b, t, 0)),
        out_shape=jax.ShapeDtypeStruct((B, N, cout), jnp.float32),
    )(*xs, *flat_ws)


def _interp_ids(S, T):
    return (np.arange(T) * S) // T


def kernel(input, params):
    B, N, _ = input.shape
    coords = input[..., :3]

    fcs = (params["fc_start"]["W"].T, params["fc_start"]["b"][None, :],
           params["bn_start"]["g"][None, :], params["bn_start"]["be"][None, :])
    x = _chain_call([input], [fcs], ["lrelu02"])

    douts = [16, 64, 128, 256]
    x_stack, np_stack = [], []
    decim, target_N = 1, N
    for i in range(4):
        x = _lfa_dense_call(x, _knn_feats_call(coords), params[f"enc{i}"],
                            douts[i])
        x_stack.append(x)
        np_stack.append(target_N)
        target_N = max(1, N // (decim * 4))
        perm = jax.random.permutation(jax.random.key(100 + i),
                                      coords.shape[1])[:target_N]
        coords = coords[:, perm, :]
        x = x[:, perm, :]
        decim *= 4

    for i in range(4):
        up = np_stack.pop()
        x = jnp.take(x, _interp_ids(x.shape[1], up), axis=1)
        skip = x_stack.pop()
        if skip.shape[1] != target_N:
            skip = jnp.take(skip, _interp_ids(skip.shape[1], up), axis=1)
        x = _chain_call([x, skip], [_mlp_t(params[f"dec{i}"])], ["relu"])

    x = _chain_call([x], [_mlp_t(params["fc_end0"]),
                          _mlp_t(params["fc_end1"]),
                          _mlp_t(params["fc_end2"])],
                    ["relu", "relu", "relu"])
    return jnp.transpose(x, (0, 2, 1))


# speedup vs baseline: 13.5340x; 1.0416x over previous
import functools
from functools import partial

import jax
import jax.numpy as jnp
import numpy as np
from jax import lax
from jax.experimental import pallas as pl
from jax.experimental.pallas import tpu as pltpu
from jax.experimental.pallas import tpu_sc as plsc

K_NN = 16
BIG = 3.0e38

_DOT = partial(jax.lax.dot_general, precision=jax.lax.Precision.HIGHEST,
               preferred_element_type=jnp.float32)


def _bmm(a, b):
    return jax.lax.dot_general(a.astype(jnp.bfloat16), b.astype(jnp.bfloat16),
                               (((1,), (0,)), ((), ())),
                               preferred_element_type=jnp.float32)


def _d2_body(c_ref, out_ref, *, T, N):
    t = pl.program_id(1)
    c = c_ref[0]
    q = c_ref[0, pl.ds(t * T, T), :]
    cross = jax.lax.dot_general(q.astype(jnp.bfloat16), c.astype(jnp.bfloat16),
                                (((1,), (1,)), ((), ())),
                                preferred_element_type=jnp.float32)
    q0, q1, q2 = q[:, 0:1], q[:, 1:2], q[:, 2:3]
    n2q = (q0 * q0 + q1 * q1) + q2 * q2
    n2c = _DOT(jnp.ones((1, 3), jnp.float32), c * c,
               (((1,), (1,)), ((), ())))
    out_ref[0] = (n2q + n2c) - 2.0 * cross


def _d2_call(coords):
    B, N, _ = coords.shape
    T = min(N, 512)
    return pl.pallas_call(
        partial(_d2_body, T=T, N=N),
        grid=(B, N // T),
        in_specs=[pl.BlockSpec((1, N, 3), lambda b, t: (b, 0, 0))],
        out_specs=pl.BlockSpec((1, T, N), lambda b, t: (b, t, 0)),
        out_shape=jax.ShapeDtypeStruct((B, N, N), jnp.float32),
    )(coords)


CH = 32


@functools.lru_cache(None)
def _sc_knn_make(B, N):
    NC = N // CH
    G = N // 16
    NW = 32
    mesh = plsc.VectorSubcoreMesh(core_axis_name="c", subcore_axis_name="s")

    @functools.partial(
        pl.kernel, mesh=mesh,
        compiler_params=pltpu.CompilerParams(needs_layout_passes=False),
        out_type=jax.ShapeDtypeStruct((B, G, K_NN * 256), jnp.float32),
        scratch_types=[
            pltpu.VMEM((16 * N,), jnp.float32),
            pltpu.VMEM((NC * 16,), jnp.float32),
            pltpu.VMEM((K_NN * 16 * 16,), jnp.float32),
            pltpu.VMEM((N,), jnp.float32),
            pltpu.VMEM((N,), jnp.float32),
            pltpu.VMEM((N,), jnp.float32),
        ],
    )
    def knn(d2_hbm, cx_hbm, cy_hbm, cz_hbm, out_hbm, dt, M, ob, tx, ty, tz):
        wid = lax.axis_index("s") * 2 + lax.axis_index("c")
        lanes = lax.broadcasted_iota(jnp.int32, (16,), 0)
        lanesN = lanes * N
        bigv = jnp.full((16,), BIG, jnp.float32)

        def gat(j):
            return plsc.load_gather(dt, [lanesN + j])

        for b in range(B):
            pltpu.sync_copy(cx_hbm.at[b], tx)
            pltpu.sync_copy(cy_hbm.at[b], ty)
            pltpu.sync_copy(cz_hbm.at[b], tz)
            trip = (G - wid + NW - 1) // NW

            def group_body(gi, _, b=b):
                g = wid + gi * NW
                n0 = g * 16
                pltpu.sync_copy(d2_hbm.at[b, pl.ds(n0 * N, 16 * N)], dt)
                qx = tx[pl.ds(n0, 16)]
                qy = ty[pl.ds(n0, 16)]
                qz = tz[pl.ds(n0, 16)]

                def build(c, _):
                    j0 = jnp.full((16,), c * CH, jnp.int32)
                    vs = [gat(j0 + ((lanes + p) & (CH - 1)))
                          for p in range(CH)]
                    while len(vs) > 1:
                        vs = [jnp.minimum(vs[i], vs[i + 1])
                              for i in range(0, len(vs), 2)]
                    M[pl.ds(c * 16, 16)] = vs[0]
                    return 0

                lax.fori_loop(0, NC, build, 0)

                def extract(k, _):
                    nodes = [(M[pl.ds(c * 16, 16)],
                              jnp.full((16,), c, jnp.int32))
                             for c in range(NC)]
                    while len(nodes) > 1:
                        nxt = []
                        for i in range(0, len(nodes), 2):
                            (va, ia), (vb, ib) = nodes[i], nodes[i + 1]
                            mk = vb < va
                            nxt.append((jnp.where(mk, vb, va),
                                        jnp.where(mk, ib, ia)))
                        nodes = nxt
                    mval, midx = nodes[0]
                    base = midx * CH
                    tri = []
                    for p in range(CH):
                        jv = base + ((lanes + p) & (CH - 1))
                        tri.append((gat(jv), jv,
                                    jnp.full((16,), jnp.inf, jnp.float32)))
                    while len(tri) > 1:
                        nxt = []
                        for i in range(0, len(tri), 2):
                            (va, ja, sa), (vb, jb, sb) = tri[i], tri[i + 1]
                            mk = (vb < va) | ((vb == va) & (jb < ja))
                            nxt.append((jnp.where(mk, vb, va),
                                        jnp.where(mk, jb, ja),
                                        jnp.minimum(jnp.minimum(sa, sb),
                                                    jnp.where(mk, va, vb))))
                        tri = nxt
                    cur, jbest, m2 = tri[0]
                    plsc.store_scatter(dt, [lanesN + jbest], bigv)
                    plsc.store_scatter(M, [midx * 16 + lanes], m2)
                    nbx = plsc.load_gather(tx, [jbest])
                    nby = plsc.load_gather(ty, [jbest])
                    nbz = plsc.load_gather(tz, [jbest])
                    dist = jnp.maximum(cur, 0.0)
                    kf = jnp.full((16,), jnp.maximum(k - 1, 0), jnp.int32)
                    zv = jnp.zeros((16,), jnp.float32)
                    vals = [qx, qy, qz, nbx, nby, nbz, qx - nbx, qy - nby,
                            qz - nbz, dist, zv, zv, zv, zv, zv, zv]
                    for ch, val in enumerate(vals):
                        plsc.store_scatter(
                            ob, [kf * 256 + lanes * 16 + ch], val)
                    return 0

                lax.fori_loop(0, K_NN + 1, extract, 0)
                pltpu.sync_copy(ob, out_hbm.at[b, g])
                return 0

            lax.fori_loop(0, trip, group_body, 0)

    return knn


def _knn_feats_tc_body(c_ref, out_ref, *, T, N):
    t = pl.program_id(1)
    c = c_ref[0]
    q = c_ref[0, pl.ds(t * T, T), :]
    cross = jax.lax.dot_general(q.astype(jnp.bfloat16), c.astype(jnp.bfloat16),
                                (((1,), (1,)), ((), ())),
                                preferred_element_type=jnp.float32)
    q0, q1, q2 = q[:, 0:1], q[:, 1:2], q[:, 2:3]
    n2q = (q0 * q0 + q1 * q1) + q2 * q2
    n2c = _DOT(jnp.ones((1, 3), jnp.float32), c * c, (((1,), (1,)), ((), ())))
    d2 = (n2q + n2c) - 2.0 * cross
    col = jax.lax.broadcasted_iota(jnp.int32, (T, N), 1)
    zpad = jnp.zeros((T, 6), jnp.float32)

    def step(j, d2m):
        m = jnp.min(d2m, axis=1, keepdims=True)
        idxc = jnp.where(d2m == m, col, N)
        im = jnp.min(idxc, axis=1, keepdims=True)
        oh = (col == im).astype(jnp.float32)
        nb = _DOT(oh, c, (((1,), (0,)), ((), ())))
        dist = jnp.maximum(m, 0.0)
        feats = jnp.concatenate([q, nb, q - nb, dist, zpad], axis=1)

        @pl.when(j > 0)
        def _():
            out_ref[0, jnp.maximum(j - 1, 0)] = feats

        return jnp.where(col == im, BIG, d2m)

    jax.lax.fori_loop(0, K_NN + 1, step, d2)


def _knn_feats_call(coords):
    B, N, _ = coords.shape
    if N < 128:
        T = min(N, 128)
        return pl.pallas_call(
            partial(_knn_feats_tc_body, T=T, N=N),
            grid=(B, N // T),
            in_specs=[pl.BlockSpec((1, N, 3), lambda b, t: (b, 0, 0))],
            out_specs=pl.BlockSpec((1, K_NN, T, 16),
                                   lambda b, t: (b, 0, t, 0)),
            out_shape=jax.ShapeDtypeStruct((B, K_NN, N, 16), jnp.float32),
        )(coords)
    d2 = _d2_call(coords).reshape(B, N * N)
    f = _sc_knn_make(B, N)(d2, coords[..., 0], coords[..., 1], coords[..., 2])
    return f.reshape(B, N // 16, K_NN, 16, 16).transpose(0, 2, 1, 3, 4
                                                         ).reshape(B, K_NN, N, 16)


def _relu(v):
    return jnp.maximum(v, 0.0)


def _lrelu02(v):
    return jnp.where(v > 0, v, 0.2 * v)


def _smlp(a, w, b, g, be, act):
    return act((_bmm(a, w[...]) + b[...]) * g[...] + be[...])


def _attpool(cat, w, b, K, T, C):
    attn = (_bmm(cat.reshape(K * T, C), w[...]) + b[...]).reshape(K, T, C)
    attn = attn - jnp.max(attn, axis=0, keepdims=True)
    attn = jnp.exp(attn)
    attn = attn / jnp.sum(attn, axis=0, keepdims=True)
    return jnp.sum(attn * cat, axis=0)


def _lfa_dense_body(x_ref, f_ref, w1, b1, g1, e1, wl1, bl1, gl1, el1, wp1, bp1,
                    wmp, bmp, gmp, emp, wl2, bl2, gl2, el2, wp2, bp2,
                    w2, b2, g2, e2, wr, br, gr, er, out_ref, *, T, dout):
    h, qd = dout // 2, dout // 4
    x = x_ref[0]
    ff = f_ref[0].reshape(K_NN * T, 16)

    x1 = _smlp(x, w1, b1, g1, e1, _lrelu02)
    sf1 = _smlp(ff, wl1, bl1, gl1, el1, _relu).reshape(K_NN, T, h)
    cat1 = jnp.concatenate(
        [sf1, jnp.broadcast_to(x1[None], (K_NN, T, h))], axis=2)
    pooled = _attpool(cat1, wp1, bp1, K_NN, T, dout)

    x2 = _smlp(pooled, wmp, bmp, gmp, emp, _relu)
    sf2 = _smlp(ff, wl2, bl2, gl2, el2, _relu).reshape(K_NN, T, qd)
    cat2 = jnp.concatenate(
        [sf2, jnp.broadcast_to(x2[None], (K_NN, T, qd))], axis=2)
    pooled2 = _attpool(cat2, wp2, bp2, K_NN, T, h)

    xm = _smlp(pooled2, w2, b2, g2, e2, _relu)
    res = _smlp(x, wr, br, gr, er, _relu)
    out = jnp.concatenate([xm, res], axis=1)
    out_ref[0] = jnp.where(out > 0, out, 0.01 * out)


def _mlp_t(p):
    return (p["W"].T, p["b"][None, :], p["g"][None, :], p["be"][None, :])


def _lin_t(p):
    return p["W"].T, p["b"][None, :]


def _lfa_dense_call(x, feats, p, dout):
    B, N, din = x.shape
    T = min(N, 512 if dout <= 128 else 128)
    def _pad16(wb):
        w, b, g, be = wb
        return (jnp.concatenate([w, jnp.zeros((6, w.shape[1]), w.dtype)], 0),
                b, g, be)

    ws = [*_mlp_t(p["mlp1"]), *_pad16(_mlp_t(p["lse1"])), *_lin_t(p["pool1"]),
          *_mlp_t(p["mlp_pool1"]), *_pad16(_mlp_t(p["lse2"])),
          *_lin_t(p["pool2"]), *_mlp_t(p["mlp2"]), *_mlp_t(p["residual"])]
    wspecs = [pl.BlockSpec(w.shape, lambda b, t: (0, 0)) for w in ws]
    return pl.pallas_call(
        partial(_lfa_dense_body, T=T, dout=dout),
        grid=(B, N // T),
        in_specs=[pl.BlockSpec((1, T, din), lambda b, t: (b, t, 0)),
                  pl.BlockSpec((1, K_NN, T, 16), lambda b, t: (b, 0, t, 0)),
                  *wspecs],
        out_specs=pl.BlockSpec((1, T, 2 * dout), lambda b, t: (b, t, 0)),
        out_shape=jax.ShapeDtypeStruct((B, N, 2 * dout), jnp.float32),
    )(x, feats, *ws)


def _chain_body(*refs, n_in, layers, acts):
    in_refs = refs[:n_in]
    w_refs = refs[n_in:-1]
    out_ref = refs[-1]
    hcat = jnp.concatenate([r[0] for r in in_refs], axis=1)
    for i in range(layers):
        w, b, g, be = w_refs[4 * i:4 * i + 4]
        act = _relu if acts[i] == "relu" else _lrelu02
        hcat = _smlp(hcat, w, b, g, be, act)
    out_ref[0] = hcat


def _chain_call(xs, wbs, acts):
    B, N = xs[0].shape[0], xs[0].shape[1]
    T = min(N, 512)
    cout = wbs[-1][0].shape[1]
    flat_ws = [a for wb in wbs for a in wb]
    in_specs = [pl.BlockSpec((1, T, x.shape[2]), lambda b, t: (b, t, 0))
                for x in xs]
    in_specs += [pl.BlockSpec(w.shape, lambda b, t: (0, 0)) for w in flat_ws]
    return pl.pallas_call(
        partial(_chain_body, n_in=len(xs), layers=len(wbs), acts=acts),
        grid=(B, N // T),
        in_specs=in_specs,
        out_specs=pl.BlockSpec((1, T, cout), lambda b, t: (b, t, 0)),
        out_shape=jax.ShapeDtypeStruct((B, N, cout), jnp.float32),
    )(*xs, *flat_ws)


def _interp_ids(S, T):
    return (np.arange(T) * S) // T


def kernel(input, params):
    B, N, _ = input.shape
    coords = input[..., :3]

    fcs = (params["fc_start"]["W"].T, params["fc_start"]["b"][None, :],
           params["bn_start"]["g"][None, :], params["bn_start"]["be"][None, :])
    x = _chain_call([input], [fcs], ["lrelu02"])

    douts = [16, 64, 128, 256]
    x_stack, np_stack = [], []
    decim, target_N = 1, N
    for i in range(4):
        x = _lfa_dense_call(x, _knn_feats_call(coords), params[f"enc{i}"],
                            douts[i])
        x_stack.append(x)
        np_stack.append(target_N)
        target_N = max(1, N // (decim * 4))
        perm = jax.random.permutation(jax.random.key(100 + i),
                                      coords.shape[1])[:target_N]
        coords = coords[:, perm, :]
        x = x[:, perm, :]
        decim *= 4

    for i in range(4):
        up = np_stack.pop()
        x = jnp.take(x, _interp_ids(x.shape[1], up), axis=1)
        skip = x_stack.pop()
        if skip.shape[1] != target_N:
            skip = jnp.take(skip, _interp_ids(skip.shape[1], up), axis=1)
        x = _chain_call([x, skip], [_mlp_t(params[f"dec{i}"])], ["relu"])

    x = _chain_call([x], [_mlp_t(params["fc_end0"]),
                          _mlp_t(params["fc_end1"]),
                          _mlp_t(params["fc_end2"])],
                    ["relu", "relu", "relu"])
    return jnp.transpose(x, (0, 2, 1))
